# Initial kernel scaffold; baseline (speedup 1.0000x reference)
import jax
import jax.numpy as jnp
from jax.experimental import pallas as pl


def kernel(hash_feature, hash_out, cls_out, target, sim_matrix_last, sim_matrix_now, count_matrix, epoch):
    raise NotImplementedError("write your pallas kernel here")


# speedup vs baseline: 217.7280x; 217.7280x over previous
import jax
import jax.numpy as jnp
from jax.experimental import pallas as pl
from jax.experimental.pallas import tpu as pltpu

B = 4096
D = 128
DH = 64
C = 101
CP = 128
GAMM = 1.0
ALPH = 0.01
TI = 512
NB = B // TI
OFFS = tuple(i * NB - i * (i - 1) // 2 for i in range(NB))
NT = NB * (NB + 1) // 2


def _ij(t):
    i = jnp.int32(0)
    for k in range(1, NB):
        i = i + (t >= OFFS[k]).astype(jnp.int32)
    j = t - (i * NB - i * (i - 1) // 2) + i
    return i, j


def _dot_nt(a, b):
    return jax.lax.dot_general(a, b, (((1,), (1,)), ((), ())),
                               preferred_element_type=jnp.float32)


def _dot_tn(a, b):
    return jax.lax.dot_general(a, b, (((0,), (0,)), ((), ())),
                               preferred_element_type=jnp.float32)


def _dot(a, b):
    return jax.lax.dot_general(a, b, (((1,), (0,)), ((), ())),
                               preferred_element_type=jnp.float32)


def _softplus(x):
    return jnp.maximum(x, 0.0) + jnp.log1p(jnp.exp(-jnp.abs(x)))


def _correct_flags(x, tgt):
    m = jnp.max(x, axis=1, keepdims=True)
    lane = jax.lax.broadcasted_iota(jnp.int32, x.shape, 1)
    firstmax = jnp.min(jnp.where(x == m, lane, CP), axis=1)
    return (firstmax == tgt).astype(jnp.float32)


def _body(ho_i, ho_j, hf_i, hf_j, cls_i, cls_j, tg_i, tg_j, ho6, tg0,
          simL, simn, cntm,
          sim_out, cnt_out, stats_out,
          accA, accC, s_cnt, s_P, s_N, s_dP, s_dN, s_nll):
    t = pl.program_id(0)
    i, j = _ij(t)
    diag = i == j

    @pl.when(t == 0)
    def _init():
        accA[:, :] = jnp.zeros((CP, CP), jnp.float32)
        accC[:, :] = jnp.zeros((CP, CP), jnp.float32)
        for r in (s_cnt, s_P, s_N, s_dP, s_dN, s_nll):
            r[:, :] = jnp.zeros((1, 1), jnp.float32)

    ti = tg_i[0, 0, :]
    tj = tg_j[0, 0, :]

    theta = _dot_nt(ho_i[:, :], ho_j[:, :]) * 0.5
    sim = (ti[:, None] == tj[None, :]).astype(jnp.float32)
    pl_t = _softplus(theta) - sim * theta
    w = jnp.where(diag, 1.0, 2.0)
    sum_pos = jnp.sum(sim)
    sum_all_pl = jnp.sum(pl_t)
    sum_pos_pl = jnp.sum(pl_t * sim)
    s_cnt[:, :] = s_cnt[:, :] + (w * sum_pos).reshape(1, 1)
    s_P[:, :] = s_P[:, :] + (w * sum_pos_pl).reshape(1, 1)
    s_N[:, :] = s_N[:, :] + (w * (sum_all_pl - sum_pos_pl)).reshape(1, 1)

    @pl.when(diag)
    def _diag_terms():
        r2 = jax.lax.broadcasted_iota(jnp.int32, (TI, TI), 0)
        c2 = jax.lax.broadcasted_iota(jnp.int32, (TI, TI), 1)
        eye = (r2 == c2).astype(jnp.float32)
        d_all = jnp.sum(pl_t * eye)
        d_pos = jnp.sum(pl_t * sim * eye)
        s_dP[:, :] = s_dP[:, :] + d_pos.reshape(1, 1)
        s_dN[:, :] = s_dN[:, :] + (d_all - d_pos).reshape(1, 1)
        x = cls_i[:, :]
        m = jnp.max(x, axis=1, keepdims=True)
        lse = m[:, 0] + jnp.log(jnp.sum(jnp.exp(x - m), axis=1))
        ohi_l = (ti[:, None] ==
                 jax.lax.broadcasted_iota(jnp.int32, (TI, CP), 1)).astype(jnp.float32)
        logit = jnp.sum(x * ohi_l, axis=1)
        s_nll[:, :] = s_nll[:, :] + jnp.sum(lse - logit).reshape(1, 1)

    corr_i = _correct_flags(cls_i[:, :], ti)
    corr_j = _correct_flags(cls_j[:, :], tj)
    xi = hf_i[:, :]
    xj = hf_j[:, :]
    ni = jnp.sqrt(jnp.sum(xi * xi, axis=1, keepdims=True))
    nj = jnp.sqrt(jnp.sum(xj * xj, axis=1, keepdims=True))
    xni = xi / jnp.maximum(ni, 1e-12)
    xnj = xj / jnp.maximum(nj, 1e-12)
    simil = _dot_nt(xni, xnj)
    r2 = jax.lax.broadcasted_iota(jnp.int32, (TI, TI), 0)
    c2 = jax.lax.broadcasted_iota(jnp.int32, (TI, TI), 1)
    tri = jnp.where(diag, (c2 > r2).astype(jnp.float32), 1.0)
    pm = corr_i[:, None] * corr_j[None, :] * tri
    vals = simil * pm
    ohi = (ti[:, None] ==
           jax.lax.broadcasted_iota(jnp.int32, (TI, CP), 1)).astype(jnp.float32)
    ohj = (tj[:, None] ==
           jax.lax.broadcasted_iota(jnp.int32, (TI, CP), 1)).astype(jnp.float32)
    accA[:, :] = accA[:, :] + _dot_tn(ohi, _dot(vals, ohj))
    accC[:, :] = accC[:, :] + _dot_tn(ohi, _dot(pm, ohj))

    @pl.when(t == NT - 1)
    def _final():
        h6 = ho6[:, :]
        th6 = _dot_nt(h6, h6) * 0.5
        t8 = tg0[0, 0, :8]
        oh6 = (t8[:, None] ==
               jax.lax.broadcasted_iota(jnp.int32, (8, CP), 1)).astype(jnp.float32)
        g6 = _dot_nt(_dot(oh6, simL[:, :]), oh6)
        r8 = jax.lax.broadcasted_iota(jnp.int32, (8, 8), 0)
        c8 = jax.lax.broadcasted_iota(jnp.int32, (8, 8), 1)
        valid = ((r8 < 6) & (c8 < 6)).astype(jnp.float32)
        eye8 = (r8 == c8).astype(jnp.float32)
        sim_old = (t8[:, None] == t8[None, :]).astype(jnp.float32)
        pos_new = (g6 == 1.0).astype(jnp.float32)
        sp6 = _softplus(th6)
        pl_old = sp6 - sim_old * th6
        pl_new = sp6 - g6 * th6
        d_pos_term = (pl_new * pos_new - pl_old * sim_old) * valid
        d_neg_term = (pl_new * (1.0 - pos_new) - pl_old * (1.0 - sim_old)) * valid
        cntP = jnp.sum(s_cnt[:, :]) + jnp.sum((pos_new - sim_old) * valid)
        P = jnp.sum(s_P[:, :]) + jnp.sum(d_pos_term)
        Nn = jnp.sum(s_N[:, :]) + jnp.sum(d_neg_term)
        dPd = jnp.sum(s_dP[:, :]) + jnp.sum(d_pos_term * eye8)
        dNd = jnp.sum(s_dN[:, :]) + jnp.sum(d_neg_term * eye8)

        Bf = jnp.float32(B)
        S1 = cntP - Bf
        S0 = Bf * Bf - cntP
        S0 = jnp.where(S0 == 0.0, 1.0, S0)
        S1 = jnp.where(S1 == 0.0, 1.0, S1)
        S = S0 + S1
        total = (P - dPd) * (S / S1) + (Nn - dNd) * (S / S0)
        hash_loss = total / 2.0 / (Bf * (Bf - 1.0) / 2.0)
        cls_loss = jnp.sum(s_nll[:, :]) / Bf
        loss = GAMM * cls_loss + ALPH * hash_loss

        eyeC = (jax.lax.broadcasted_iota(jnp.int32, (CP, CP), 0) ==
                jax.lax.broadcasted_iota(jnp.int32, (CP, CP), 1))
        A = accA[:, :]
        sim_out[:, :] = simn[:, :] + A + A.T - jnp.where(eyeC, A, 0.0)
        Cn = accC[:, :]
        cnt_out[:, :] = cntm[:, :] + Cn + Cn.T - jnp.where(eyeC, Cn, 0.0)
        lane = jax.lax.broadcasted_iota(jnp.int32, (1, CP), 1)
        stats_out[:, :] = (hash_loss * (lane == 0) + cls_loss * (lane == 1)
                           + loss * (lane == 2)).astype(jnp.float32)


def kernel(hash_feature, hash_out, cls_out, target, sim_matrix_last,
           sim_matrix_now, count_matrix, epoch):
    del epoch
    cls_pad = jnp.pad(cls_out, ((0, 0), (0, CP - C)), constant_values=-1e30)
    simL_pad = jnp.pad(sim_matrix_last, ((0, CP - C), (0, CP - C)))
    simn_pad = jnp.pad(sim_matrix_now, ((0, CP - C), (0, CP - C)))
    cnt_pad = jnp.pad(count_matrix, ((0, CP - C), (0, CP - C)))
    tgt3 = target.astype(jnp.int32).reshape(NB, 1, TI)

    def im_i(t):
        i, _ = _ij(t)
        return (i, 0)

    def im_j(t):
        _, j = _ij(t)
        return (j, 0)

    def im_ti(t):
        i, _ = _ij(t)
        return (i, 0, 0)

    def im_tj(t):
        _, j = _ij(t)
        return (j, 0, 0)

    const2 = lambda t: (0, 0)
    const3 = lambda t: (0, 0, 0)

    grid_spec = pltpu.PrefetchScalarGridSpec(
        num_scalar_prefetch=0,
        grid=(NT,),
        in_specs=[
            pl.BlockSpec((TI, DH), im_i),
            pl.BlockSpec((TI, DH), im_j),
            pl.BlockSpec((TI, D), im_i),
            pl.BlockSpec((TI, D), im_j),
            pl.BlockSpec((TI, CP), im_i),
            pl.BlockSpec((TI, CP), im_j),
            pl.BlockSpec((1, 1, TI), im_ti),
            pl.BlockSpec((1, 1, TI), im_tj),
            pl.BlockSpec((8, DH), const2),
            pl.BlockSpec((1, 1, TI), const3),
            pl.BlockSpec((CP, CP), const2),
            pl.BlockSpec((CP, CP), const2),
            pl.BlockSpec((CP, CP), const2),
        ],
        out_specs=[
            pl.BlockSpec((CP, CP), const2),
            pl.BlockSpec((CP, CP), const2),
            pl.BlockSpec((1, CP), const2),
        ],
        scratch_shapes=[
            pltpu.VMEM((CP, CP), jnp.float32),
            pltpu.VMEM((CP, CP), jnp.float32),
        ] + [pltpu.VMEM((1, 1), jnp.float32) for _ in range(6)],
    )

    sim_p, cnt_p, stats = pl.pallas_call(
        _body,
        grid_spec=grid_spec,
        out_shape=[
            jax.ShapeDtypeStruct((CP, CP), jnp.float32),
            jax.ShapeDtypeStruct((CP, CP), jnp.float32),
            jax.ShapeDtypeStruct((1, CP), jnp.float32),
        ],
    )(hash_out, hash_out, hash_feature, hash_feature, cls_pad, cls_pad,
      tgt3, tgt3, hash_out, tgt3, simL_pad, simn_pad, cnt_pad)

    return (sim_p[:C, :C], cnt_p[:C, :C],
            stats[0, 0], stats[0, 1], stats[0, 2])
